# baseline (speedup 1.0000x reference)
import numpy as np

import jax
import jax.numpy as jnp
from jax import lax
from jax.experimental import pallas as pl
from jax.experimental.pallas import tpu as pltpu
from jax.experimental.pallas import tpu_sc as plsc

N = 10000
E = 320000
D = 128
ALPHA = 0.2

NROW = 10016
DUMMY = N
NC, NS = 2, 16
NW = NC * NS
K = 32
EPT = 10240
E_PAD = NW * EPT
CPT = EPT // K
NO = CPT // 16
CROWS = CPT + 8
ROWS_COM = 536
ROWS_EXT = 96

VPERM = np.zeros(D, np.int32)
for _j in range(4):
    for _i in range(16):
        VPERM[16 * _j + _i] = 32 * _j + _i
        VPERM[64 + 16 * _j + _i] = 32 * _j + 16 + _i


def _pack_words(h):
    b = lax.bitcast_convert_type(h.astype(jnp.bfloat16), jnp.uint16)
    lo = b[:, : D // 2].astype(jnp.uint32)
    hi = b[:, D // 2:].astype(jnp.uint32) << 16
    return (lo | hi).astype(jnp.int32)


def _proj_body(xk_ref, xv_ref, wk_ref, bk_ref, wv_ref, bv_ref, hkv_ref):
    hk = jnp.dot(xk_ref[...], wk_ref[...],
                 preferred_element_type=jnp.float32) + bk_ref[...]
    hv = jnp.dot(xv_ref[...], wv_ref[...],
                 preferred_element_type=jnp.float32) + bv_ref[...]
    hkv_ref[:, : D // 2] = _pack_words(hk)
    hkv_ref[:, D // 2:] = _pack_words(hv)


def _lane_shuffle(a, idx):
    return lax.gather(
        a, idx[:, None],
        dimension_numbers=lax.GatherDimensionNumbers(
            offset_dims=(), collapsed_slice_dims=(0,), start_index_map=(0,)),
        slice_sizes=(1,),
        mode=lax.GatherScatterMode.PROMISE_IN_BOUNDS)


def _sc_body(hkv_hbm, ec_hbm, out_hbm, den_hbm,
             idxq, rows_s, rows_d, msg, wbuf, denom,
             acc, s0, s1, s2, s3, t0, t1, u0, u1):
    cid = lax.axis_index("c")
    sid = lax.axis_index("s")
    wid = cid * NS + sid
    sems = [s0, s1, s2, s3]
    dsems = [t0, t1]
    ssems = [u0, u1]
    base_row = sid * (ROWS_COM + ROWS_EXT)

    def zrow(r, _):
        for p in range(2):
            for c in range(D // 16):
                msg[p, r, pl.ds(c * 16, 16)] = jnp.zeros((16,), jnp.float32)
        return _
    lax.fori_loop(0, K, zrow, None)
    pltpu.async_copy(msg.at[0], acc.at[pl.ds(base_row, K)], ssems[0])
    pltpu.async_copy(msg.at[1], acc.at[pl.ds(base_row + K, K)], ssems[1])
    for b in range(2, ROWS_COM // K):
        pltpu.sync_copy(msg.at[0], acc.at[pl.ds(base_row + b * K, K)])
    pltpu.sync_copy(msg.at[0, pl.ds(0, ROWS_COM % K)],
                    acc.at[pl.ds(base_row + (ROWS_COM // K) * K,
                                 ROWS_COM % K)])

    @pl.when(sid < NS - 1)
    def _():
        for b in range(ROWS_EXT // K):
            pltpu.sync_copy(msg.at[0],
                            acc.at[pl.ds(base_row + ROWS_COM + b * K, K)])

    def zden(r, _):
        denom[pl.ds(r * 16, 16)] = jnp.zeros((16,), jnp.float32)
        return _
    lax.fori_loop(0, NROW // 16, zden, None)
    plsc.subcore_barrier()

    cb = wid * CROWS
    lanes = lax.iota(jnp.int32, 16)
    UNPACK = dict(format=plsc.PackFormat.INTERLEAVED)

    def bf2(x):
        return plsc.unpack(plsc.bitcast(x, jnp.bfloat16), **UNPACK)

    def make_compute(slot):
        slot2 = slot % 2

        def group(e0):
            e_idx = lanes + e0

            def dot_body(d, carry):
                d_vec, s_acc = carry
                sv = plsc.load_gather(rows_s.at[slot], [e_idx, d_vec])
                dv = plsc.load_gather(rows_d.at[slot2], [e_idx, d_vec])
                sa, sb = bf2(sv)
                da, db = bf2(dv)
                s_acc = s_acc + sa * da + sb * db
                return d_vec + 1, s_acc

            _, s = lax.fori_loop(
                0, D // 2, dot_body,
                (jnp.zeros((16,), jnp.int32), jnp.zeros((16,), jnp.float32)),
                unroll=4)
            w = jnp.exp(s)
            wbuf[pl.ds(e0, 16)] = w

            def val_body(l, _):
                e = e0 + l
                wl = _lane_shuffle(w, jnp.full((16,), l, jnp.int32))
                for j in range(4):
                    va, vb = bf2(rows_s[slot, e, pl.ds(64 + 16 * j, 16)])
                    msg[slot2, e, pl.ds(32 * j, 16)] = va * wl
                    msg[slot2, e, pl.ds(32 * j + 16, 16)] = vb * wl
                return _

            lax.fori_loop(0, 16, val_body, None, unroll=2)

        def compute():
            for e0 in range(0, K, 16):
                group(e0)
        return compute

    computes = [make_compute(slot) for slot in range(4)]

    def denacc(p, j):
        for g in range(K // 16):
            dvec = idxq[p, j, 1, pl.ds(g * 16, 16)]
            wvec = wbuf[pl.ds(g * 16, 16)]
            for l in range(16):
                plsc.addupdate_scatter(denom, [dvec], wvec, mask=lanes == l)

    def issue_s(p, j, slot):
        pltpu.async_copy(hkv_hbm.at[idxq.at[p, j, 0]], rows_s.at[slot],
                         sems[slot])

    def wait_s(p, j, slot):
        pltpu.make_async_copy(hkv_hbm.at[idxq.at[p, j, 0]], rows_s.at[slot],
                              sems[slot]).wait()

    def issue_d(p, j, slot2):
        pltpu.async_copy(hkv_hbm.at[idxq.at[p, j, 1]], rows_d.at[slot2],
                         dsems[slot2])

    def wait_d(p, j, slot2):
        pltpu.make_async_copy(hkv_hbm.at[idxq.at[p, j, 1]], rows_d.at[slot2],
                              dsems[slot2]).wait()

    def octet(p, tb):
        for r8 in range(8):
            slot = r8 % 4
            slot2 = r8 % 2
            wait_s(p, r8, slot)
            wait_d(p, r8, slot2)
            pltpu.make_async_copy(msg.at[slot2], acc.at[idxq.at[p, r8, 1]],
                                  ssems[slot2]).wait()
            computes[slot]()
            denacc(p, r8)
            pltpu.async_copy(msg.at[slot2], acc.at[idxq.at[p, r8, 1]],
                             ssems[slot2], add=True)
            if r8 == 1:
                off = pl.multiple_of(cb + tb + 8, 8)
                pltpu.sync_copy(ec_hbm.at[pl.ds(off, 8)], idxq.at[1 - p])
            if r8 < 4:
                issue_s(p, r8 + 4, slot)
            else:
                issue_s(1 - p, r8 - 4, slot)
            if r8 < 6:
                issue_d(p, r8 + 2, slot2)
            else:
                issue_d(1 - p, r8 - 6, slot2)

    pltpu.sync_copy(ec_hbm.at[pl.ds(pl.multiple_of(cb, 8), 8)], idxq.at[0])
    for r in range(4):
        issue_s(0, r, r)
    for r in range(2):
        issue_d(0, r, r)

    def gbody(gg, _):
        octet(0, 16 * gg)
        octet(1, 16 * gg + 8)
        return _

    lax.fori_loop(0, NO, gbody, None)
    for r in range(4):
        wait_s(0, r, r)
    for r in range(2):
        wait_d(0, r, r)
    for r in range(2):
        pltpu.make_async_copy(msg.at[r], acc.at[idxq.at[1, 6 + r, 1]],
                              ssems[r]).wait()

    pltpu.sync_copy(denom, den_hbm.at[wid])
    plsc.subcore_barrier()
    pltpu.sync_copy(acc.at[pl.ds(base_row, ROWS_COM)],
                    out_hbm.at[cid, pl.ds(base_row, ROWS_COM)])

    @pl.when(sid < NS - 1)
    def _():
        pltpu.sync_copy(acc.at[pl.ds(base_row + ROWS_COM, ROWS_EXT)],
                        out_hbm.at[cid, pl.ds(base_row + ROWS_COM, ROWS_EXT)])


def _comb_body(p_ref, den_ref, o_ref):
    v = p_ref[0] + p_ref[1]
    d = jnp.sum(den_ref[...], axis=1)
    d = jnp.where(d == 0.0, 1.0, d)
    o = v / d[:, None]
    o_ref[...] = jnp.where(o >= 0.0, o, ALPHA * o)


def kernel(X_key, X_value, edge_index, Wk, bk, Wv, bv):
    xk = X_key.reshape(N, D)
    xv = X_value.reshape(N, D)
    pad = ((0, NROW - N), (0, 0))
    xk = jnp.pad(xk, pad)
    xv = jnp.pad(xv, pad)
    bk2 = bk.reshape(1, D)
    bv2 = bv[VPERM].reshape(1, D)
    Wv2 = Wv[:, VPERM]

    RB = 2504
    grid = NROW // RB
    hkv = pl.pallas_call(
        _proj_body,
        grid=(grid,),
        in_specs=[
            pl.BlockSpec((RB, D), lambda i: (i, 0)),
            pl.BlockSpec((RB, D), lambda i: (i, 0)),
            pl.BlockSpec((D, D), lambda i: (0, 0)),
            pl.BlockSpec((1, D), lambda i: (0, 0)),
            pl.BlockSpec((D, D), lambda i: (0, 0)),
            pl.BlockSpec((1, D), lambda i: (0, 0)),
        ],
        out_specs=pl.BlockSpec((RB, D), lambda i: (i, 0)),
        out_shape=jax.ShapeDtypeStruct((NROW, D), jnp.int32),
    )(xk, xv, Wk, bk2, Wv2, bv2)

    src = edge_index[0]
    dst = edge_index[1]
    fill = jnp.full((E_PAD - E,), DUMMY, jnp.int32)
    src_c = jnp.concatenate([src, fill]).reshape(NW, CPT, K)
    dst_c = jnp.concatenate([dst, fill]).reshape(NW, CPT, K)
    ec = jnp.stack([src_c, dst_c], axis=2)
    dummy_rows = jnp.full((NW, 8, 2, K), DUMMY, jnp.int32)
    ec = jnp.concatenate([ec, dummy_rows], axis=1)
    ec = ec.reshape(NW * CROWS, 2, K)

    mesh = plsc.VectorSubcoreMesh(core_axis_name="c", subcore_axis_name="s")
    acc, den = pl.kernel(
        _sc_body,
        out_type=[
            jax.ShapeDtypeStruct((NC, NROW, D), jnp.float32),
            jax.ShapeDtypeStruct((NW, NROW), jnp.float32),
        ],
        mesh=mesh,
        compiler_params=pltpu.CompilerParams(needs_layout_passes=False),
        scratch_types=[
            pltpu.VMEM((2, 8, 2, K), jnp.int32),
            pltpu.VMEM((4, K, D), jnp.int32),
            pltpu.VMEM((2, K, D), jnp.int32),
            pltpu.VMEM((2, K, D), jnp.float32),
            pltpu.VMEM((K,), jnp.float32),
            pltpu.VMEM((NROW,), jnp.float32),
            pltpu.VMEM_SHARED((NROW, D), jnp.float32),
            pltpu.SemaphoreType.DMA,
            pltpu.SemaphoreType.DMA,
            pltpu.SemaphoreType.DMA,
            pltpu.SemaphoreType.DMA,
            pltpu.SemaphoreType.DMA,
            pltpu.SemaphoreType.DMA,
            pltpu.SemaphoreType.DMA,
            pltpu.SemaphoreType.DMA,
        ],
    )(hkv, ec)

    out = pl.pallas_call(
        _comb_body,
        grid=(grid,),
        in_specs=[
            pl.BlockSpec((NC, RB, D), lambda i: (0, i, 0)),
            pl.BlockSpec((RB, NW), lambda i: (i, 0)),
        ],
        out_specs=pl.BlockSpec((RB, D), lambda i: (i, 0)),
        out_shape=jax.ShapeDtypeStruct((NROW, D), jnp.float32),
    )(acc, den.T)

    return out[:N].reshape(1, N, D)


# speedup vs baseline: 1.0129x; 1.0129x over previous
import numpy as np

import jax
import jax.numpy as jnp
from jax import lax
from jax.experimental import pallas as pl
from jax.experimental.pallas import tpu as pltpu
from jax.experimental.pallas import tpu_sc as plsc

N = 10000
E = 320000
D = 128
ALPHA = 0.2

N_PAD = 10240
DUMMY = N
NC, NS = 2, 16
NW = NC * NS
K = 48
EPT = 10272
E_PAD = NW * EPT
CPT = EPT // K
CROWS = CPT + 2

VPERM = np.zeros(D, np.int32)
for _j in range(4):
    for _i in range(16):
        VPERM[16 * _j + _i] = 32 * _j + _i
        VPERM[64 + 16 * _j + _i] = 32 * _j + 16 + _i


def _pack_words(h):
    b = lax.bitcast_convert_type(h.astype(jnp.bfloat16), jnp.uint16)
    lo = b[:, : D // 2].astype(jnp.uint32)
    hi = b[:, D // 2:].astype(jnp.uint32) << 16
    return (lo | hi).astype(jnp.int32)


def _proj_body(xk_ref, xv_ref, wk_ref, bk_ref, wv_ref, bv_ref,
               hkv_ref, hk_ref):
    hk = jnp.dot(xk_ref[...], wk_ref[...],
                 preferred_element_type=jnp.float32) + bk_ref[...]
    hv = jnp.dot(xv_ref[...], wv_ref[...],
                 preferred_element_type=jnp.float32) + bv_ref[...]
    hkv_ref[:, : D // 2] = _pack_words(hk)
    hkv_ref[:, D // 2:] = _pack_words(hv)
    hk_ref[...] = hk


def _lane_shuffle(a, idx):
    return lax.gather(
        a, idx[:, None],
        dimension_numbers=lax.GatherDimensionNumbers(
            offset_dims=(), collapsed_slice_dims=(0,), start_index_map=(0,)),
        slice_sizes=(1,),
        mode=lax.GatherScatterMode.PROMISE_IN_BOUNDS)


def _sc_body(hkv_hbm, hk_hbm, ec_hbm, out_hbm, den_hbm,
             idxc, rows_src, rows_dst, msg, wbuf, denom,
             acc, sa1, sa2, sb1, sb2):
    cid = lax.axis_index("c")
    sid = lax.axis_index("s")
    wid = cid * NS + sid
    rows_per_tile = N_PAD // NS

    def zrow(r, _):
        for c in range(D // 16):
            msg[r, pl.ds(c * 16, 16)] = jnp.zeros((16,), jnp.float32)
        return _
    lax.fori_loop(0, K, zrow, None)
    for b in range(rows_per_tile // K):
        pltpu.sync_copy(msg, acc.at[pl.ds(sid * rows_per_tile + b * K, K)])
    rem = rows_per_tile % K
    pltpu.sync_copy(
        msg.at[pl.ds(0, rem)],
        acc.at[pl.ds(sid * rows_per_tile + (rows_per_tile // K) * K, rem)])

    def zden(r, _):
        denom[pl.ds(r * 16, 16)] = jnp.zeros((16,), jnp.float32)
        return _
    lax.fori_loop(0, N_PAD // 16, zden, None)
    plsc.subcore_barrier()

    cb = wid * CROWS
    lanes = lax.iota(jnp.int32, 16)
    UNPACK = dict(format=plsc.PackFormat.INTERLEAVED)

    def bf2(x):
        return plsc.unpack(plsc.bitcast(x, jnp.bfloat16), **UNPACK)

    def make_compute(p):
        def group(e0):
            e_idx = lanes + e0

            def dot_body(d, carry):
                d_vec, s_acc = carry
                sv = plsc.load_gather(rows_src.at[p], [e_idx, d_vec])
                da = plsc.load_gather(rows_dst.at[p], [e_idx, d_vec])
                db = plsc.load_gather(rows_dst.at[p], [e_idx, d_vec + 64])
                sa, sb = bf2(sv)
                s_acc = s_acc + sa * da + sb * db
                return d_vec + 1, s_acc

            _, s = lax.fori_loop(
                0, D // 2, dot_body,
                (jnp.zeros((16,), jnp.int32), jnp.zeros((16,), jnp.float32)),
                unroll=4)
            w = jnp.exp(s)
            wbuf[pl.ds(e0, 16)] = w

            def val_body(l, _):
                e = e0 + l
                wl = _lane_shuffle(w, jnp.full((16,), l, jnp.int32))
                for j in range(4):
                    va, vb = bf2(rows_src[p, e, pl.ds(64 + 16 * j, 16)])
                    msg[e, pl.ds(32 * j, 16)] = va * wl
                    msg[e, pl.ds(32 * j + 16, 16)] = vb * wl
                return _

            lax.fori_loop(0, 16, val_body, None, unroll=2)

        def compute():
            for e0 in range(0, K, 16):
                group(e0)
        return compute

    computes = [make_compute(p) for p in range(2)]

    def denacc(p):
        for g in range(K // 16):
            dvec = idxc[p, 1, pl.ds(g * 16, 16)]
            wvec = wbuf[pl.ds(g * 16, 16)]
            for l in range(16):
                plsc.addupdate_scatter(denom, [dvec], wvec, mask=lanes == l)

    def issue(p, row):
        pltpu.sync_copy(ec_hbm.at[row], idxc.at[p])
        pltpu.async_copy(hkv_hbm.at[idxc.at[p, 0]], rows_src.at[p], sa1 if p == 0 else sb1)
        pltpu.async_copy(hk_hbm.at[idxc.at[p, 1]], rows_dst.at[p], sa2 if p == 0 else sb2)

    def wait(p):
        pltpu.make_async_copy(hkv_hbm.at[idxc.at[p, 0]], rows_src.at[p],
                              sa1 if p == 0 else sb1).wait()
        pltpu.make_async_copy(hk_hbm.at[idxc.at[p, 1]], rows_dst.at[p],
                              sa2 if p == 0 else sb2).wait()

    def do_chunk(p):
        computes[p]()
        denacc(p)
        pltpu.sync_copy(msg, acc.at[idxc.at[p, 1]], add=True)

    issue(0, cb)

    def pair(i, _):
        ta = 2 * i
        issue(1, cb + ta + 1)
        wait(0)
        do_chunk(0)
        issue(0, cb + ta + 2)
        wait(1)
        do_chunk(1)
        return _

    lax.fori_loop(0, CPT // 2, pair, None)
    wait(0)

    pltpu.sync_copy(denom, den_hbm.at[wid])
    plsc.subcore_barrier()
    r0 = sid * rows_per_tile
    pltpu.sync_copy(acc.at[pl.ds(r0, rows_per_tile)],
                    out_hbm.at[cid, pl.ds(r0, rows_per_tile)])


def _comb_body(p_ref, den_ref, o_ref):
    v = p_ref[0] + p_ref[1]
    d = jnp.sum(den_ref[...], axis=0)
    d = jnp.where(d == 0.0, 1.0, d)
    o = v / d[:, None]
    o_ref[...] = jnp.where(o >= 0.0, o, ALPHA * o)


def kernel(X_key, X_value, edge_index, Wk, bk, Wv, bv):
    xk = X_key.reshape(N, D)
    xv = X_value.reshape(N, D)
    pad = ((0, N_PAD - N), (0, 0))
    xk = jnp.pad(xk, pad)
    xv = jnp.pad(xv, pad)
    bk2 = bk.reshape(1, D)
    bv2 = bv[VPERM].reshape(1, D)
    Wv2 = Wv[:, VPERM]

    RB = 2560
    grid = N_PAD // RB
    hkv, hk = pl.pallas_call(
        _proj_body,
        grid=(grid,),
        in_specs=[
            pl.BlockSpec((RB, D), lambda i: (i, 0)),
            pl.BlockSpec((RB, D), lambda i: (i, 0)),
            pl.BlockSpec((D, D), lambda i: (0, 0)),
            pl.BlockSpec((1, D), lambda i: (0, 0)),
            pl.BlockSpec((D, D), lambda i: (0, 0)),
            pl.BlockSpec((1, D), lambda i: (0, 0)),
        ],
        out_specs=[
            pl.BlockSpec((RB, D), lambda i: (i, 0)),
            pl.BlockSpec((RB, D), lambda i: (i, 0)),
        ],
        out_shape=[
            jax.ShapeDtypeStruct((N_PAD, D), jnp.int32),
            jax.ShapeDtypeStruct((N_PAD, D), jnp.float32),
        ],
    )(xk, xv, Wk, bk2, Wv2, bv2)

    src = edge_index[0]
    dst = edge_index[1]
    fill = jnp.full((E_PAD - E,), DUMMY, jnp.int32)
    src_c = jnp.concatenate([src, fill]).reshape(NW, CPT, K)
    dst_c = jnp.concatenate([dst, fill]).reshape(NW, CPT, K)
    ec = jnp.stack([src_c, dst_c], axis=2)
    dummy_rows = jnp.full((NW, 2, 2, K), DUMMY, jnp.int32)
    ec = jnp.concatenate([ec, dummy_rows], axis=1)
    ec = ec.reshape(NW * CROWS, 2, K)

    mesh = plsc.VectorSubcoreMesh(core_axis_name="c", subcore_axis_name="s")
    acc, den = pl.kernel(
        _sc_body,
        out_type=[
            jax.ShapeDtypeStruct((NC, N_PAD, D), jnp.float32),
            jax.ShapeDtypeStruct((NW, N_PAD), jnp.float32),
        ],
        mesh=mesh,
        compiler_params=pltpu.CompilerParams(needs_layout_passes=False),
        scratch_types=[
            pltpu.VMEM((2, 2, K), jnp.int32),
            pltpu.VMEM((2, K, D), jnp.int32),
            pltpu.VMEM((2, K, D), jnp.float32),
            pltpu.VMEM((K, D), jnp.float32),
            pltpu.VMEM((K,), jnp.float32),
            pltpu.VMEM((N_PAD,), jnp.float32),
            pltpu.VMEM_SHARED((N_PAD, D), jnp.float32),
            pltpu.SemaphoreType.DMA,
            pltpu.SemaphoreType.DMA,
            pltpu.SemaphoreType.DMA,
            pltpu.SemaphoreType.DMA,
        ],
    )(hkv, hk, ec)

    out = pl.pallas_call(
        _comb_body,
        grid=(grid,),
        in_specs=[
            pl.BlockSpec((NC, RB, D), lambda i: (0, i, 0)),
            pl.BlockSpec((NW, RB), lambda i: (0, i)),
        ],
        out_specs=pl.BlockSpec((RB, D), lambda i: (i, 0)),
        out_shape=jax.ShapeDtypeStruct((N_PAD, D), jnp.float32),
    )(acc, den)

    return out[:N].reshape(1, N, D)


# speedup vs baseline: 1.9783x; 1.9531x over previous
import numpy as np

import jax
import jax.numpy as jnp
from jax import lax
from jax.experimental import pallas as pl
from jax.experimental.pallas import tpu as pltpu
from jax.experimental.pallas import tpu_sc as plsc

N = 10000
E = 320000
D = 128
ALPHA = 0.2

N_PAD = 10240
DUMMY = N
NC, NS = 2, 16
NW = NC * NS
K = 48
EPT = 10272
E_PAD = NW * EPT
CPT = EPT // K
CROWS = CPT + 2

VPERM = np.zeros(D, np.int32)
for _j in range(4):
    for _i in range(16):
        VPERM[16 * _j + _i] = 32 * _j + _i
        VPERM[64 + 16 * _j + _i] = 32 * _j + 16 + _i


def _pack_words(h):
    b = lax.bitcast_convert_type(h.astype(jnp.bfloat16), jnp.uint16)
    lo = b[:, : D // 2].astype(jnp.uint32)
    hi = b[:, D // 2:].astype(jnp.uint32) << 16
    return (lo | hi).astype(jnp.int32)


def _proj_body(xk_ref, xv_ref, wk_ref, bk_ref, wv_ref, bv_ref,
               hkv_ref, hk_ref):
    hk = jnp.dot(xk_ref[...], wk_ref[...],
                 preferred_element_type=jnp.float32) + bk_ref[...]
    hv = jnp.dot(xv_ref[...], wv_ref[...],
                 preferred_element_type=jnp.float32) + bv_ref[...]
    hkv_ref[:, : D // 2] = _pack_words(hk)
    hkv_ref[:, D // 2:] = _pack_words(hv)
    hk_ref[...] = hk


def _lane_shuffle(a, idx):
    return lax.gather(
        a, idx[:, None],
        dimension_numbers=lax.GatherDimensionNumbers(
            offset_dims=(), collapsed_slice_dims=(0,), start_index_map=(0,)),
        slice_sizes=(1,),
        mode=lax.GatherScatterMode.PROMISE_IN_BOUNDS)


def _sc_body(hkv_hbm, hk_hbm, ec_hbm, out_hbm, den_hbm,
             idxc, rows_src, rows_dst, msg, wbuf, denom,
             acc, sa1, sa2, sb1, sb2):
    cid = lax.axis_index("c")
    sid = lax.axis_index("s")
    wid = cid * NS + sid
    rows_per_tile = N_PAD // NS

    def zrow(r, _):
        for c in range(D // 16):
            msg[r, pl.ds(c * 16, 16)] = jnp.zeros((16,), jnp.float32)
        return _
    lax.fori_loop(0, K, zrow, None)
    for b in range(rows_per_tile // K):
        pltpu.sync_copy(msg, acc.at[pl.ds(sid * rows_per_tile + b * K, K)])
    rem = rows_per_tile % K
    pltpu.sync_copy(
        msg.at[pl.ds(0, rem)],
        acc.at[pl.ds(sid * rows_per_tile + (rows_per_tile // K) * K, rem)])

    def zden(r, _):
        denom[pl.ds(r * 16, 16)] = jnp.zeros((16,), jnp.float32)
        return _
    lax.fori_loop(0, N_PAD // 16, zden, None)
    plsc.subcore_barrier()

    cb = wid * CROWS
    lanes = lax.iota(jnp.int32, 16)
    UNPACK = dict(format=plsc.PackFormat.INTERLEAVED)

    def bf2(x):
        return plsc.unpack(plsc.bitcast(x, jnp.bfloat16), **UNPACK)

    def make_compute(p):
        def group(e0):
            e_idx = lanes + e0

            def dot_body(d, carry):
                d_vec, s_acc = carry
                col = jnp.bitwise_and(d_vec, 63)
                sv = plsc.load_gather(rows_src.at[p], [e_idx, col])
                da = plsc.load_gather(rows_dst.at[p], [e_idx, col])
                db = plsc.load_gather(rows_dst.at[p], [e_idx, col + 64])
                sa, sb = bf2(sv)
                s_acc = s_acc + sa * da + sb * db
                return d_vec + 1, s_acc

            _, s = lax.fori_loop(
                0, D // 2, dot_body,
                (lanes, jnp.zeros((16,), jnp.float32)),
                unroll=4)
            w = jnp.exp(s)
            wbuf[pl.ds(e0, 16)] = w

            def val_body(l, _):
                e = e0 + l
                wl = _lane_shuffle(w, jnp.full((16,), l, jnp.int32))
                for j in range(4):
                    va, vb = bf2(rows_src[p, e, pl.ds(64 + 16 * j, 16)])
                    msg[e, pl.ds(32 * j, 16)] = va * wl
                    msg[e, pl.ds(32 * j + 16, 16)] = vb * wl
                return _

            lax.fori_loop(0, 16, val_body, None, unroll=2)

        def compute():
            for e0 in range(0, K, 16):
                group(e0)
        return compute

    computes = [make_compute(p) for p in range(2)]

    def denacc(p):
        for g in range(K // 16):
            dvec = idxc[p, 1, pl.ds(g * 16, 16)]
            wvec = wbuf[pl.ds(g * 16, 16)]
            for l in range(16):
                plsc.addupdate_scatter(denom, [dvec], wvec, mask=lanes == l)

    def issue(p, row):
        pltpu.sync_copy(ec_hbm.at[row], idxc.at[p])
        pltpu.async_copy(hkv_hbm.at[idxc.at[p, 0]], rows_src.at[p], sa1 if p == 0 else sb1)
        pltpu.async_copy(hk_hbm.at[idxc.at[p, 1]], rows_dst.at[p], sa2 if p == 0 else sb2)

    def wait(p):
        pltpu.make_async_copy(hkv_hbm.at[idxc.at[p, 0]], rows_src.at[p],
                              sa1 if p == 0 else sb1).wait()
        pltpu.make_async_copy(hk_hbm.at[idxc.at[p, 1]], rows_dst.at[p],
                              sa2 if p == 0 else sb2).wait()

    def do_chunk(p):
        computes[p]()
        denacc(p)
        pltpu.sync_copy(msg, acc.at[idxc.at[p, 1]], add=True)

    issue(0, cb)

    def pair(i, _):
        ta = 2 * i
        issue(1, cb + ta + 1)
        wait(0)
        do_chunk(0)
        issue(0, cb + ta + 2)
        wait(1)
        do_chunk(1)
        return _

    lax.fori_loop(0, CPT // 2, pair, None)
    wait(0)

    pltpu.sync_copy(denom, den_hbm.at[wid])
    plsc.subcore_barrier()
    r0 = sid * rows_per_tile
    pltpu.sync_copy(acc.at[pl.ds(r0, rows_per_tile)],
                    out_hbm.at[cid, pl.ds(r0, rows_per_tile)])


def _comb_body(p_ref, den_ref, o_ref):
    v = p_ref[0] + p_ref[1]
    d = jnp.sum(den_ref[...], axis=0)
    d = jnp.where(d == 0.0, 1.0, d)
    o = v / d[:, None]
    o_ref[...] = jnp.where(o >= 0.0, o, ALPHA * o)


def kernel(X_key, X_value, edge_index, Wk, bk, Wv, bv):
    xk = X_key.reshape(N, D)
    xv = X_value.reshape(N, D)
    pad = ((0, N_PAD - N), (0, 0))
    xk = jnp.pad(xk, pad)
    xv = jnp.pad(xv, pad)
    bk2 = bk.reshape(1, D)
    bv2 = bv[VPERM].reshape(1, D)
    Wv2 = Wv[:, VPERM]

    RB = 2560
    grid = N_PAD // RB
    hkv, hk = pl.pallas_call(
        _proj_body,
        grid=(grid,),
        in_specs=[
            pl.BlockSpec((RB, D), lambda i: (i, 0)),
            pl.BlockSpec((RB, D), lambda i: (i, 0)),
            pl.BlockSpec((D, D), lambda i: (0, 0)),
            pl.BlockSpec((1, D), lambda i: (0, 0)),
            pl.BlockSpec((D, D), lambda i: (0, 0)),
            pl.BlockSpec((1, D), lambda i: (0, 0)),
        ],
        out_specs=[
            pl.BlockSpec((RB, D), lambda i: (i, 0)),
            pl.BlockSpec((RB, D), lambda i: (i, 0)),
        ],
        out_shape=[
            jax.ShapeDtypeStruct((N_PAD, D), jnp.int32),
            jax.ShapeDtypeStruct((N_PAD, D), jnp.float32),
        ],
    )(xk, xv, Wk, bk2, Wv2, bv2)

    src = edge_index[0]
    dst = edge_index[1]
    fill = jnp.full((E_PAD - E,), DUMMY, jnp.int32)
    src_c = jnp.concatenate([src, fill]).reshape(NW, CPT, K)
    dst_c = jnp.concatenate([dst, fill]).reshape(NW, CPT, K)
    ec = jnp.stack([src_c, dst_c], axis=2)
    dummy_rows = jnp.full((NW, 2, 2, K), DUMMY, jnp.int32)
    ec = jnp.concatenate([ec, dummy_rows], axis=1)
    ec = ec.reshape(NW * CROWS, 2, K)

    mesh = plsc.VectorSubcoreMesh(core_axis_name="c", subcore_axis_name="s")
    acc, den = pl.kernel(
        _sc_body,
        out_type=[
            jax.ShapeDtypeStruct((NC, N_PAD, D), jnp.float32),
            jax.ShapeDtypeStruct((NW, N_PAD), jnp.float32),
        ],
        mesh=mesh,
        compiler_params=pltpu.CompilerParams(needs_layout_passes=False),
        scratch_types=[
            pltpu.VMEM((2, 2, K), jnp.int32),
            pltpu.VMEM((2, K, D), jnp.int32),
            pltpu.VMEM((2, K, D), jnp.float32),
            pltpu.VMEM((K, D), jnp.float32),
            pltpu.VMEM((K,), jnp.float32),
            pltpu.VMEM((N_PAD,), jnp.float32),
            pltpu.VMEM_SHARED((N_PAD, D), jnp.float32),
            pltpu.SemaphoreType.DMA,
            pltpu.SemaphoreType.DMA,
            pltpu.SemaphoreType.DMA,
            pltpu.SemaphoreType.DMA,
        ],
    )(hkv, hk, ec)

    out = pl.pallas_call(
        _comb_body,
        grid=(grid,),
        in_specs=[
            pl.BlockSpec((NC, RB, D), lambda i: (0, i, 0)),
            pl.BlockSpec((NW, RB), lambda i: (0, i)),
        ],
        out_specs=pl.BlockSpec((RB, D), lambda i: (i, 0)),
        out_shape=jax.ShapeDtypeStruct((N_PAD, D), jnp.float32),
    )(acc, den)

    return out[:N].reshape(1, N, D)


# speedup vs baseline: 2.0062x; 1.0141x over previous
import numpy as np

import jax
import jax.numpy as jnp
from jax import lax
from jax.experimental import pallas as pl
from jax.experimental.pallas import tpu as pltpu
from jax.experimental.pallas import tpu_sc as plsc

N = 10000
E = 320000
D = 128
ALPHA = 0.2

N_PAD = 10240
DUMMY = N
NC, NS = 2, 16
NW = NC * NS
K = 48
EPT = 10272
E_PAD = NW * EPT
CPT = EPT // K
CROWS = CPT + 2

VPERM = np.zeros(D, np.int32)
for _j in range(4):
    for _i in range(16):
        VPERM[16 * _j + _i] = 32 * _j + _i
        VPERM[64 + 16 * _j + _i] = 32 * _j + 16 + _i


def _pack_words(h):
    b = lax.bitcast_convert_type(h.astype(jnp.bfloat16), jnp.uint16)
    lo = b[:, : D // 2].astype(jnp.uint32)
    hi = b[:, D // 2:].astype(jnp.uint32) << 16
    return (lo | hi).astype(jnp.int32)


def _proj_body(xk_ref, xv_ref, wk_ref, bk_ref, wv_ref, bv_ref,
               hkv_ref, hk_ref):
    hk = jnp.dot(xk_ref[...], wk_ref[...],
                 preferred_element_type=jnp.float32) + bk_ref[...]
    hv = jnp.dot(xv_ref[...], wv_ref[...],
                 preferred_element_type=jnp.float32) + bv_ref[...]
    hkv_ref[:, : D // 2] = _pack_words(hk)
    hkv_ref[:, D // 2:] = _pack_words(hv)
    hk_ref[...] = hk


def _lane_shuffle(a, idx):
    return lax.gather(
        a, idx[:, None],
        dimension_numbers=lax.GatherDimensionNumbers(
            offset_dims=(), collapsed_slice_dims=(0,), start_index_map=(0,)),
        slice_sizes=(1,),
        mode=lax.GatherScatterMode.PROMISE_IN_BOUNDS)


def _sc_body(hkv_hbm, hk_hbm, ec_hbm, out_hbm, den_hbm,
             idxc, sidx, rows_src, rows_dst, msg, wbuf, denom,
             acc, sa1, sa2, sb1, sb2, scs):
    cid = lax.axis_index("c")
    sid = lax.axis_index("s")
    wid = cid * NS + sid
    rows_per_tile = N_PAD // NS

    def zrow(r, _):
        for c in range(D // 16):
            msg[r, pl.ds(c * 16, 16)] = jnp.zeros((16,), jnp.float32)
        return _
    lax.fori_loop(0, K, zrow, None)
    for p in range(2):
        for g in range(K // 16):
            sidx[p, pl.ds(16 * g, 16)] = jnp.zeros((16,), jnp.int32)
    pltpu.async_copy(msg, acc.at[pl.ds(sid * rows_per_tile, K)], scs)
    for b in range(1, rows_per_tile // K):
        pltpu.sync_copy(msg, acc.at[pl.ds(sid * rows_per_tile + b * K, K)])
    rem = rows_per_tile % K
    pltpu.sync_copy(
        msg.at[pl.ds(0, rem)],
        acc.at[pl.ds(sid * rows_per_tile + (rows_per_tile // K) * K, rem)])

    def zden(r, _):
        denom[pl.ds(r * 16, 16)] = jnp.zeros((16,), jnp.float32)
        return _
    lax.fori_loop(0, N_PAD // 16, zden, None)
    plsc.subcore_barrier()

    cb = wid * CROWS
    lanes = lax.iota(jnp.int32, 16)
    UNPACK = dict(format=plsc.PackFormat.INTERLEAVED)

    def bf2(x):
        return plsc.unpack(plsc.bitcast(x, jnp.bfloat16), **UNPACK)

    def make_compute(p):
        def group(e0):
            e_idx = lanes + e0

            def dot_body(d, carry):
                d_vec, s_acc = carry
                col = jnp.bitwise_and(d_vec, 63)
                sv = plsc.load_gather(rows_src.at[p], [e_idx, col])
                da = plsc.load_gather(rows_dst.at[p], [e_idx, col])
                db = plsc.load_gather(rows_dst.at[p], [e_idx, col + 64])
                sa, sb = bf2(sv)
                s_acc = s_acc + sa * da + sb * db
                return d_vec + 1, s_acc

            _, s = lax.fori_loop(
                0, D // 2, dot_body,
                (lanes, jnp.zeros((16,), jnp.float32)),
                unroll=4)
            w = jnp.exp(s)
            wbuf[pl.ds(e0, 16)] = w

            def val_body(l, _):
                e = e0 + l
                wl = _lane_shuffle(w, jnp.full((16,), l, jnp.int32))
                for j in range(4):
                    va, vb = bf2(rows_src[p, e, pl.ds(64 + 16 * j, 16)])
                    msg[e, pl.ds(32 * j, 16)] = va * wl
                    msg[e, pl.ds(32 * j + 16, 16)] = vb * wl
                return _

            lax.fori_loop(0, 16, val_body, None, unroll=2)

        def compute():
            for e0 in range(0, K, 16):
                group(e0)
        return compute

    computes = [make_compute(p) for p in range(2)]

    def denacc(p):
        for g in range(K // 16):
            dvec = idxc[p, 1, pl.ds(g * 16, 16)]
            wvec = wbuf[pl.ds(g * 16, 16)]
            for l in range(16):
                plsc.addupdate_scatter(denom, [dvec], wvec, mask=lanes == l)

    def issue(p, row):
        pltpu.sync_copy(ec_hbm.at[row], idxc.at[p])
        pltpu.async_copy(hkv_hbm.at[idxc.at[p, 0]], rows_src.at[p], sa1 if p == 0 else sb1)
        pltpu.async_copy(hk_hbm.at[idxc.at[p, 1]], rows_dst.at[p], sa2 if p == 0 else sb2)

    def wait(p):
        pltpu.make_async_copy(hkv_hbm.at[idxc.at[p, 0]], rows_src.at[p],
                              sa1 if p == 0 else sb1).wait()
        pltpu.make_async_copy(hk_hbm.at[idxc.at[p, 1]], rows_dst.at[p],
                              sa2 if p == 0 else sb2).wait()

    def do_chunk(p):
        pltpu.make_async_copy(msg, acc.at[sidx.at[p]], scs).wait()
        computes[p]()
        for g in range(K // 16):
            sidx[p, pl.ds(16 * g, 16)] = idxc[p, 1, pl.ds(16 * g, 16)]
        pltpu.async_copy(msg, acc.at[sidx.at[p]], scs, add=True)
        denacc(p)

    issue(0, cb)

    def pair(i, _):
        ta = 2 * i
        issue(1, cb + ta + 1)
        wait(0)
        do_chunk(0)
        issue(0, cb + ta + 2)
        wait(1)
        do_chunk(1)
        return _

    lax.fori_loop(0, CPT // 2, pair, None)
    wait(0)
    pltpu.make_async_copy(msg, acc.at[sidx.at[1]], scs).wait()

    pltpu.sync_copy(denom, den_hbm.at[wid])
    plsc.subcore_barrier()
    r0 = sid * rows_per_tile
    pltpu.sync_copy(acc.at[pl.ds(r0, rows_per_tile)],
                    out_hbm.at[cid, pl.ds(r0, rows_per_tile)])


def _comb_body(p_ref, den_ref, o_ref):
    v = p_ref[0] + p_ref[1]
    d = jnp.sum(den_ref[...], axis=0)
    d = jnp.where(d == 0.0, 1.0, d)
    o = v / d[:, None]
    o_ref[...] = jnp.where(o >= 0.0, o, ALPHA * o)


def kernel(X_key, X_value, edge_index, Wk, bk, Wv, bv):
    xk = X_key.reshape(N, D)
    xv = X_value.reshape(N, D)
    pad = ((0, N_PAD - N), (0, 0))
    xk = jnp.pad(xk, pad)
    xv = jnp.pad(xv, pad)
    bk2 = bk.reshape(1, D)
    bv2 = bv[VPERM].reshape(1, D)
    Wv2 = Wv[:, VPERM]

    RB = 2560
    grid = N_PAD // RB
    hkv, hk = pl.pallas_call(
        _proj_body,
        grid=(grid,),
        in_specs=[
            pl.BlockSpec((RB, D), lambda i: (i, 0)),
            pl.BlockSpec((RB, D), lambda i: (i, 0)),
            pl.BlockSpec((D, D), lambda i: (0, 0)),
            pl.BlockSpec((1, D), lambda i: (0, 0)),
            pl.BlockSpec((D, D), lambda i: (0, 0)),
            pl.BlockSpec((1, D), lambda i: (0, 0)),
        ],
        out_specs=[
            pl.BlockSpec((RB, D), lambda i: (i, 0)),
            pl.BlockSpec((RB, D), lambda i: (i, 0)),
        ],
        out_shape=[
            jax.ShapeDtypeStruct((N_PAD, D), jnp.int32),
            jax.ShapeDtypeStruct((N_PAD, D), jnp.float32),
        ],
    )(xk, xv, Wk, bk2, Wv2, bv2)

    src = edge_index[0]
    dst = edge_index[1]
    fill = jnp.full((E_PAD - E,), DUMMY, jnp.int32)
    src_c = jnp.concatenate([src, fill]).reshape(NW, CPT, K)
    dst_c = jnp.concatenate([dst, fill]).reshape(NW, CPT, K)
    ec = jnp.stack([src_c, dst_c], axis=2)
    dummy_rows = jnp.full((NW, 2, 2, K), DUMMY, jnp.int32)
    ec = jnp.concatenate([ec, dummy_rows], axis=1)
    ec = ec.reshape(NW * CROWS, 2, K)

    mesh = plsc.VectorSubcoreMesh(core_axis_name="c", subcore_axis_name="s")
    acc, den = pl.kernel(
        _sc_body,
        out_type=[
            jax.ShapeDtypeStruct((NC, N_PAD, D), jnp.float32),
            jax.ShapeDtypeStruct((NW, N_PAD), jnp.float32),
        ],
        mesh=mesh,
        compiler_params=pltpu.CompilerParams(needs_layout_passes=False),
        scratch_types=[
            pltpu.VMEM((2, 2, K), jnp.int32),
            pltpu.VMEM((2, K), jnp.int32),
            pltpu.VMEM((2, K, D), jnp.int32),
            pltpu.VMEM((2, K, D), jnp.float32),
            pltpu.VMEM((K, D), jnp.float32),
            pltpu.VMEM((K,), jnp.float32),
            pltpu.VMEM((N_PAD,), jnp.float32),
            pltpu.VMEM_SHARED((N_PAD, D), jnp.float32),
            pltpu.SemaphoreType.DMA,
            pltpu.SemaphoreType.DMA,
            pltpu.SemaphoreType.DMA,
            pltpu.SemaphoreType.DMA,
            pltpu.SemaphoreType.DMA,
        ],
    )(hkv, hk, ec)

    out = pl.pallas_call(
        _comb_body,
        grid=(grid,),
        in_specs=[
            pl.BlockSpec((NC, RB, D), lambda i: (0, i, 0)),
            pl.BlockSpec((NW, RB), lambda i: (0, i)),
        ],
        out_specs=pl.BlockSpec((RB, D), lambda i: (i, 0)),
        out_shape=jax.ShapeDtypeStruct((N_PAD, D), jnp.float32),
    )(acc, den)

    return out[:N].reshape(1, N, D)


# speedup vs baseline: 2.0859x; 1.0397x over previous
import numpy as np

import jax
import jax.numpy as jnp
from jax import lax
from jax.experimental import pallas as pl
from jax.experimental.pallas import tpu as pltpu
from jax.experimental.pallas import tpu_sc as plsc

N = 10000
E = 320000
D = 128
ALPHA = 0.2

N_PAD = 10240
DUMMY = N
NC, NS = 2, 16
NW = NC * NS
K = 48
CP0, CP1 = 184, 244
E_PAD = NW * (CP0 + CP1) // 2 * K
CROWS = CP1 + 2

VPERM = np.zeros(D, np.int32)
for _j in range(4):
    for _i in range(16):
        VPERM[16 * _j + _i] = 32 * _j + _i
        VPERM[64 + 16 * _j + _i] = 32 * _j + 16 + _i


def _pack_words(h):
    b = lax.bitcast_convert_type(h.astype(jnp.bfloat16), jnp.uint16)
    lo = b[:, : D // 2].astype(jnp.uint32)
    hi = b[:, D // 2:].astype(jnp.uint32) << 16
    return (lo | hi).astype(jnp.int32)


def _proj_body(xk_ref, xv_ref, wk_ref, bk_ref, wv_ref, bv_ref,
               hkv_ref, hk_ref):
    hk = jnp.dot(xk_ref[...], wk_ref[...],
                 preferred_element_type=jnp.float32) + bk_ref[...]
    hv = jnp.dot(xv_ref[...], wv_ref[...],
                 preferred_element_type=jnp.float32) + bv_ref[...]
    hkv_ref[:, : D // 2] = _pack_words(hk)
    hkv_ref[:, D // 2:] = _pack_words(hv)
    hk_ref[...] = hk


def _lane_shuffle(a, idx):
    return lax.gather(
        a, idx[:, None],
        dimension_numbers=lax.GatherDimensionNumbers(
            offset_dims=(), collapsed_slice_dims=(0,), start_index_map=(0,)),
        slice_sizes=(1,),
        mode=lax.GatherScatterMode.PROMISE_IN_BOUNDS)


def _sc_body(hkv_hbm, hk_hbm, ec_hbm, out_hbm, den_hbm,
             idxc, sidx, rows_src, rows_dst, msg, wbuf, denom,
             acc, sa1, sa2, sb1, sb2, scs):
    cid = lax.axis_index("c")
    sid = lax.axis_index("s")
    wid = cid * NS + sid
    rows_per_tile = N_PAD // NS

    def zrow(r, _):
        for c in range(D // 16):
            msg[r, pl.ds(c * 16, 16)] = jnp.zeros((16,), jnp.float32)
        return _
    lax.fori_loop(0, K, zrow, None)
    for p in range(2):
        for g in range(K // 16):
            sidx[p, pl.ds(16 * g, 16)] = jnp.zeros((16,), jnp.int32)
    pltpu.async_copy(msg, acc.at[pl.ds(sid * rows_per_tile, K)], scs)
    for b in range(1, rows_per_tile // K):
        pltpu.sync_copy(msg, acc.at[pl.ds(sid * rows_per_tile + b * K, K)])
    rem = rows_per_tile % K
    pltpu.sync_copy(
        msg.at[pl.ds(0, rem)],
        acc.at[pl.ds(sid * rows_per_tile + (rows_per_tile // K) * K, rem)])

    def zden(r, _):
        denom[pl.ds(r * 16, 16)] = jnp.zeros((16,), jnp.float32)
        return _
    lax.fori_loop(0, N_PAD // 16, zden, None)
    plsc.subcore_barrier()

    cb = wid * CROWS
    lanes = lax.iota(jnp.int32, 16)
    UNPACK = dict(format=plsc.PackFormat.INTERLEAVED)

    def bf2(x):
        return plsc.unpack(plsc.bitcast(x, jnp.bfloat16), **UNPACK)

    def make_compute(p):
        def group(e0):
            e_idx = lanes + e0

            def dot_body(d, carry):
                d_vec, s_acc = carry
                col = jnp.bitwise_and(d_vec, 63)
                sv = plsc.load_gather(rows_src.at[p], [e_idx, col])
                da = plsc.load_gather(rows_dst.at[p], [e_idx, col])
                db = plsc.load_gather(rows_dst.at[p], [e_idx, col + 64])
                sa, sb = bf2(sv)
                s_acc = s_acc + sa * da + sb * db
                return d_vec + 1, s_acc

            _, s = lax.fori_loop(
                0, D // 2, dot_body,
                (lanes, jnp.zeros((16,), jnp.float32)),
                unroll=4)
            w = jnp.exp(s)
            wbuf[pl.ds(e0, 16)] = w

            def val_body(l, _):
                e = e0 + l
                wl = _lane_shuffle(w, jnp.full((16,), l, jnp.int32))
                for j in range(4):
                    va, vb = bf2(rows_src[p, e, pl.ds(64 + 16 * j, 16)])
                    msg[e, pl.ds(32 * j, 16)] = va * wl
                    msg[e, pl.ds(32 * j + 16, 16)] = vb * wl
                return _

            lax.fori_loop(0, 16, val_body, None, unroll=2)

        def compute():
            for e0 in range(0, K, 16):
                group(e0)
        return compute

    computes = [make_compute(p) for p in range(2)]

    def denacc(p):
        for g in range(K // 16):
            dvec = idxc[p, 1, pl.ds(g * 16, 16)]
            wvec = wbuf[pl.ds(g * 16, 16)]
            for l in range(16):
                plsc.addupdate_scatter(denom, [dvec], wvec, mask=lanes == l)

    def issue(p, row):
        pltpu.sync_copy(ec_hbm.at[row], idxc.at[p])
        pltpu.async_copy(hkv_hbm.at[idxc.at[p, 0]], rows_src.at[p], sa1 if p == 0 else sb1)
        pltpu.async_copy(hk_hbm.at[idxc.at[p, 1]], rows_dst.at[p], sa2 if p == 0 else sb2)

    def wait(p):
        pltpu.make_async_copy(hkv_hbm.at[idxc.at[p, 0]], rows_src.at[p],
                              sa1 if p == 0 else sb1).wait()
        pltpu.make_async_copy(hk_hbm.at[idxc.at[p, 1]], rows_dst.at[p],
                              sa2 if p == 0 else sb2).wait()

    def do_chunk(p):
        pltpu.make_async_copy(msg, acc.at[sidx.at[p]], scs).wait()
        computes[p]()
        for g in range(K // 16):
            sidx[p, pl.ds(16 * g, 16)] = idxc[p, 1, pl.ds(16 * g, 16)]
        pltpu.async_copy(msg, acc.at[sidx.at[p]], scs, add=True)
        denacc(p)

    issue(0, cb)

    def pair(i, _):
        ta = 2 * i
        issue(1, cb + ta + 1)
        wait(0)
        do_chunk(0)
        issue(0, cb + ta + 2)
        wait(1)
        do_chunk(1)
        return _

    npairs = jnp.where(cid == 0, CP0 // 2, CP1 // 2)
    lax.fori_loop(0, npairs, pair, None)
    wait(0)
    pltpu.make_async_copy(msg, acc.at[sidx.at[1]], scs).wait()

    pltpu.sync_copy(denom, den_hbm.at[wid])
    plsc.subcore_barrier()
    r0 = sid * rows_per_tile
    pltpu.sync_copy(acc.at[pl.ds(r0, rows_per_tile)],
                    out_hbm.at[cid, pl.ds(r0, rows_per_tile)])


def _comb_body(p_ref, den_ref, o_ref):
    v = p_ref[0] + p_ref[1]
    d = jnp.sum(den_ref[...], axis=0)
    d = jnp.where(d == 0.0, 1.0, d)
    o = v / d[:, None]
    o_ref[...] = jnp.where(o >= 0.0, o, ALPHA * o)


def kernel(X_key, X_value, edge_index, Wk, bk, Wv, bv):
    xk = X_key.reshape(N, D)
    xv = X_value.reshape(N, D)
    pad = ((0, N_PAD - N), (0, 0))
    xk = jnp.pad(xk, pad)
    xv = jnp.pad(xv, pad)
    bk2 = bk.reshape(1, D)
    bv2 = bv[VPERM].reshape(1, D)
    Wv2 = Wv[:, VPERM]

    RB = 2560
    grid = N_PAD // RB
    hkv, hk = pl.pallas_call(
        _proj_body,
        grid=(grid,),
        in_specs=[
            pl.BlockSpec((RB, D), lambda i: (i, 0)),
            pl.BlockSpec((RB, D), lambda i: (i, 0)),
            pl.BlockSpec((D, D), lambda i: (0, 0)),
            pl.BlockSpec((1, D), lambda i: (0, 0)),
            pl.BlockSpec((D, D), lambda i: (0, 0)),
            pl.BlockSpec((1, D), lambda i: (0, 0)),
        ],
        out_specs=[
            pl.BlockSpec((RB, D), lambda i: (i, 0)),
            pl.BlockSpec((RB, D), lambda i: (i, 0)),
        ],
        out_shape=[
            jax.ShapeDtypeStruct((N_PAD, D), jnp.int32),
            jax.ShapeDtypeStruct((N_PAD, D), jnp.float32),
        ],
    )(xk, xv, Wk, bk2, Wv2, bv2)

    src = edge_index[0]
    dst = edge_index[1]
    fill = jnp.full((E_PAD - E,), DUMMY, jnp.int32)
    e0 = NS * CP0 * K

    def chunked(x):
        xp = jnp.concatenate([x, fill])
        a = xp[:e0].reshape(NS, CP0, K)
        a = jnp.concatenate(
            [a, jnp.full((NS, CROWS - CP0, K), DUMMY, jnp.int32)], axis=1)
        b = xp[e0:].reshape(NS, CP1, K)
        b = jnp.concatenate(
            [b, jnp.full((NS, CROWS - CP1, K), DUMMY, jnp.int32)], axis=1)
        return jnp.concatenate([a, b], axis=0)

    ec = jnp.stack([chunked(src), chunked(dst)], axis=2)
    ec = ec.reshape(NW * CROWS, 2, K)

    mesh = plsc.VectorSubcoreMesh(core_axis_name="c", subcore_axis_name="s")
    acc, den = pl.kernel(
        _sc_body,
        out_type=[
            jax.ShapeDtypeStruct((NC, N_PAD, D), jnp.float32),
            jax.ShapeDtypeStruct((NW, N_PAD), jnp.float32),
        ],
        mesh=mesh,
        compiler_params=pltpu.CompilerParams(needs_layout_passes=False),
        scratch_types=[
            pltpu.VMEM((2, 2, K), jnp.int32),
            pltpu.VMEM((2, K), jnp.int32),
            pltpu.VMEM((2, K, D), jnp.int32),
            pltpu.VMEM((2, K, D), jnp.float32),
            pltpu.VMEM((K, D), jnp.float32),
            pltpu.VMEM((K,), jnp.float32),
            pltpu.VMEM((N_PAD,), jnp.float32),
            pltpu.VMEM_SHARED((N_PAD, D), jnp.float32),
            pltpu.SemaphoreType.DMA,
            pltpu.SemaphoreType.DMA,
            pltpu.SemaphoreType.DMA,
            pltpu.SemaphoreType.DMA,
            pltpu.SemaphoreType.DMA,
        ],
    )(hkv, hk, ec)

    out = pl.pallas_call(
        _comb_body,
        grid=(grid,),
        in_specs=[
            pl.BlockSpec((NC, RB, D), lambda i: (0, i, 0)),
            pl.BlockSpec((NW, RB), lambda i: (0, i)),
        ],
        out_specs=pl.BlockSpec((RB, D), lambda i: (i, 0)),
        out_shape=jax.ShapeDtypeStruct((N_PAD, D), jnp.float32),
    )(acc, den)

    return out[:N].reshape(1, N, D)


# speedup vs baseline: 2.4118x; 1.1563x over previous
import numpy as np

import jax
import jax.numpy as jnp
from jax import lax
from jax.experimental import pallas as pl
from jax.experimental.pallas import tpu as pltpu
from jax.experimental.pallas import tpu_sc as plsc

N = 10000
E = 320000
D = 128
ALPHA = 0.2

N_PAD = 10240
DUMMY = N
NC, NS = 2, 16
NW = NC * NS
K = 48
CP0, CP1 = 180, 238
E_PAD = NW * (CP0 + CP1) // 2 * K
CROWS = CP1 + 2

VPERM = np.zeros(D, np.int32)
for _j in range(4):
    for _i in range(16):
        VPERM[16 * _j + _i] = 32 * _j + _i
        VPERM[64 + 16 * _j + _i] = 32 * _j + 16 + _i


def _pack_words(h):
    b = lax.bitcast_convert_type(h.astype(jnp.bfloat16), jnp.uint16)
    lo = b[:, : D // 2].astype(jnp.uint32)
    hi = b[:, D // 2:].astype(jnp.uint32) << 16
    return (lo | hi).astype(jnp.int32)


def _proj_body(xk_ref, xv_ref, wk_ref, bk_ref, wv_ref, bv_ref,
               hkv_ref, hk_ref):
    hk = jnp.dot(xk_ref[...], wk_ref[...],
                 preferred_element_type=jnp.float32) + bk_ref[...]
    hv = jnp.dot(xv_ref[...], wv_ref[...],
                 preferred_element_type=jnp.float32) + bv_ref[...]
    hkv_ref[:, : D // 2] = _pack_words(hk)
    hkv_ref[:, D // 2:] = _pack_words(hv)
    hk_ref[...] = hk


def _lane_shuffle(a, idx):
    return lax.gather(
        a, idx[:, None],
        dimension_numbers=lax.GatherDimensionNumbers(
            offset_dims=(), collapsed_slice_dims=(0,), start_index_map=(0,)),
        slice_sizes=(1,),
        mode=lax.GatherScatterMode.PROMISE_IN_BOUNDS)


def _sc_body(hkv_hbm, hk_hbm, ec_hbm, out_hbm, den_hbm,
             idxc, sidx, rows_src, rows_dst, msg, wbuf, denom,
             acc, sa1, sa2, sb1, sb2, scs):
    cid = lax.axis_index("c")
    sid = lax.axis_index("s")
    wid = cid * NS + sid
    rows_per_tile = N_PAD // NS

    def zrow(r, _):
        for c in range(D // 16):
            msg[r, pl.ds(c * 16, 16)] = jnp.zeros((16,), jnp.float32)
        return _
    lax.fori_loop(0, K, zrow, None)
    for p in range(2):
        for g in range(K // 16):
            sidx[p, pl.ds(16 * g, 16)] = jnp.zeros((16,), jnp.int32)
    pltpu.async_copy(msg, acc.at[pl.ds(sid * rows_per_tile, K)], scs)
    for b in range(1, rows_per_tile // K):
        pltpu.sync_copy(msg, acc.at[pl.ds(sid * rows_per_tile + b * K, K)])
    rem = rows_per_tile % K
    pltpu.sync_copy(
        msg.at[pl.ds(0, rem)],
        acc.at[pl.ds(sid * rows_per_tile + (rows_per_tile // K) * K, rem)])

    def zden(r, _):
        denom[pl.ds(r * 16, 16)] = jnp.zeros((16,), jnp.float32)
        return _
    lax.fori_loop(0, N_PAD // 16, zden, None)
    plsc.subcore_barrier()

    cb = wid * CROWS
    lanes = lax.iota(jnp.int32, 16)
    UNPACK = dict(format=plsc.PackFormat.INTERLEAVED)

    def bf2(x):
        return plsc.unpack(plsc.bitcast(x, jnp.bfloat16), **UNPACK)

    def make_compute(p):
        def group(e0):
            e_idx = lanes + e0

            def dot_body(d, carry):
                d_vec, s_acc = carry
                col = jnp.bitwise_and(d_vec, 63)
                sv = plsc.load_gather(rows_src.at[p], [e_idx, col])
                da = plsc.load_gather(rows_dst.at[p], [e_idx, col])
                db = plsc.load_gather(rows_dst.at[p], [e_idx, col + 64])
                sa, sb = bf2(sv)
                s_acc = s_acc + sa * da + sb * db
                return d_vec + 1, s_acc

            _, s = lax.fori_loop(
                0, D // 2, dot_body,
                (lanes, jnp.zeros((16,), jnp.float32)),
                unroll=4)
            w = jnp.exp(s)
            wbuf[pl.ds(e0, 16)] = w

            def val_body(l, _):
                e = e0 + l
                wl = _lane_shuffle(w, jnp.full((16,), l, jnp.int32))
                for j in range(4):
                    va, vb = bf2(rows_src[p, e, pl.ds(64 + 16 * j, 16)])
                    msg[e, pl.ds(32 * j, 16)] = va * wl
                    msg[e, pl.ds(32 * j + 16, 16)] = vb * wl
                return _

            lax.fori_loop(0, 16, val_body, None, unroll=2)

        def compute():
            for e0 in range(0, K, 16):
                group(e0)
        return compute

    computes = [make_compute(p) for p in range(2)]

    def denacc(p):
        for g in range(K // 16):
            dvec = idxc[p, 1, pl.ds(g * 16, 16)]
            wvec = wbuf[pl.ds(g * 16, 16)]
            for l in range(16):
                plsc.addupdate_scatter(denom, [dvec], wvec, mask=lanes == l)

    def issue(p, row):
        pltpu.sync_copy(ec_hbm.at[row], idxc.at[p])
        pltpu.async_copy(hkv_hbm.at[idxc.at[p, 0]], rows_src.at[p], sa1 if p == 0 else sb1)
        pltpu.async_copy(hk_hbm.at[idxc.at[p, 1]], rows_dst.at[p], sa2 if p == 0 else sb2)

    def wait(p):
        pltpu.make_async_copy(hkv_hbm.at[idxc.at[p, 0]], rows_src.at[p],
                              sa1 if p == 0 else sb1).wait()
        pltpu.make_async_copy(hk_hbm.at[idxc.at[p, 1]], rows_dst.at[p],
                              sa2 if p == 0 else sb2).wait()

    def do_chunk(p):
        pltpu.make_async_copy(msg, acc.at[sidx.at[p]], scs).wait()
        computes[p]()
        for g in range(K // 16):
            sidx[p, pl.ds(16 * g, 16)] = idxc[p, 1, pl.ds(16 * g, 16)]
        pltpu.async_copy(msg, acc.at[sidx.at[p]], scs, add=True)
        denacc(p)

    issue(0, cb)

    def pair(i, _):
        ta = 2 * i
        issue(1, cb + ta + 1)
        wait(0)
        do_chunk(0)
        issue(0, cb + ta + 2)
        wait(1)
        do_chunk(1)
        return _

    npairs = jnp.where(cid == 0, CP0 // 2, CP1 // 2)
    lax.fori_loop(0, npairs, pair, None)
    wait(0)
    pltpu.make_async_copy(msg, acc.at[sidx.at[1]], scs).wait()

    pltpu.sync_copy(denom, den_hbm.at[wid])
    plsc.subcore_barrier()
    r0 = sid * rows_per_tile
    pltpu.sync_copy(acc.at[pl.ds(r0, rows_per_tile)],
                    out_hbm.at[cid, pl.ds(r0, rows_per_tile)])


def _comb_body(p_ref, den_ref, o_ref):
    v = p_ref[0] + p_ref[1]
    d = jnp.sum(den_ref[...], axis=0)
    d = jnp.where(d == 0.0, 1.0, d)
    o = v / d[:, None]
    o_ref[...] = jnp.where(o >= 0.0, o, ALPHA * o)


def kernel(X_key, X_value, edge_index, Wk, bk, Wv, bv):
    xk = X_key.reshape(N, D)
    xv = X_value.reshape(N, D)
    pad = ((0, N_PAD - N), (0, 0))
    xk = jnp.pad(xk, pad)
    xv = jnp.pad(xv, pad)
    bk2 = bk.reshape(1, D)
    bv2 = bv[VPERM].reshape(1, D)
    Wv2 = Wv[:, VPERM]

    RB = 2560
    grid = N_PAD // RB
    hkv, hk = pl.pallas_call(
        _proj_body,
        grid=(grid,),
        in_specs=[
            pl.BlockSpec((RB, D), lambda i: (i, 0)),
            pl.BlockSpec((RB, D), lambda i: (i, 0)),
            pl.BlockSpec((D, D), lambda i: (0, 0)),
            pl.BlockSpec((1, D), lambda i: (0, 0)),
            pl.BlockSpec((D, D), lambda i: (0, 0)),
            pl.BlockSpec((1, D), lambda i: (0, 0)),
        ],
        out_specs=[
            pl.BlockSpec((RB, D), lambda i: (i, 0)),
            pl.BlockSpec((RB, D), lambda i: (i, 0)),
        ],
        out_shape=[
            jax.ShapeDtypeStruct((N_PAD, D), jnp.int32),
            jax.ShapeDtypeStruct((N_PAD, D), jnp.float32),
        ],
    )(xk, xv, Wk, bk2, Wv2, bv2)

    src = edge_index[0]
    dst = edge_index[1]
    fill = jnp.full((E_PAD - E,), DUMMY, jnp.int32)
    e0 = NS * CP0 * K

    def chunked(x):
        xp = jnp.concatenate([x, fill])
        a = xp[:e0].reshape(NS, CP0, K)
        a = jnp.concatenate(
            [a, jnp.full((NS, CROWS - CP0, K), DUMMY, jnp.int32)], axis=1)
        b = xp[e0:].reshape(NS, CP1, K)
        b = jnp.concatenate(
            [b, jnp.full((NS, CROWS - CP1, K), DUMMY, jnp.int32)], axis=1)
        return jnp.concatenate([a, b], axis=0)

    ec = jnp.stack([chunked(src), chunked(dst)], axis=2)
    ec = ec.reshape(NW * CROWS, 2, K)

    mesh = plsc.VectorSubcoreMesh(core_axis_name="c", subcore_axis_name="s")
    acc, den = pl.kernel(
        _sc_body,
        out_type=[
            jax.ShapeDtypeStruct((NC, N_PAD, D), jnp.float32),
            jax.ShapeDtypeStruct((NW, N_PAD), jnp.float32),
        ],
        mesh=mesh,
        compiler_params=pltpu.CompilerParams(needs_layout_passes=False),
        scratch_types=[
            pltpu.VMEM((2, 2, K), jnp.int32),
            pltpu.VMEM((2, K), jnp.int32),
            pltpu.VMEM((2, K, D), jnp.int32),
            pltpu.VMEM((2, K, D), jnp.float32),
            pltpu.VMEM((K, D), jnp.float32),
            pltpu.VMEM((K,), jnp.float32),
            pltpu.VMEM((N_PAD,), jnp.float32),
            pltpu.VMEM_SHARED((N_PAD, D), jnp.float32),
            pltpu.SemaphoreType.DMA,
            pltpu.SemaphoreType.DMA,
            pltpu.SemaphoreType.DMA,
            pltpu.SemaphoreType.DMA,
            pltpu.SemaphoreType.DMA,
        ],
    )(hkv, hk, ec)

    out = pl.pallas_call(
        _comb_body,
        grid=(grid,),
        in_specs=[
            pl.BlockSpec((NC, RB, D), lambda i: (0, i, 0)),
            pl.BlockSpec((NW, RB), lambda i: (0, i)),
        ],
        out_specs=pl.BlockSpec((RB, D), lambda i: (i, 0)),
        out_shape=jax.ShapeDtypeStruct((N_PAD, D), jnp.float32),
    )(acc, den)

    return out[:N].reshape(1, N, D)
